# baseline (speedup 1.0000x reference)
import jax
import jax.numpy as jnp
from jax.experimental import pallas as pl
from jax.experimental.pallas import tpu as pltpu

_NUM_TYPE = 1000
_BATCH = 1024
_HIST = 200
_CHUNK_SIZES = (96, 104, 96, 104, 96, 104, 96, 104, 96, 104)
_LANES = 128
_SUB = _BATCH // _LANES


def _offsets():
    offs, o = [], 0
    for s in _CHUNK_SIZES:
        offs.append(o)
        o += s
    return tuple(offs)


_CHUNK_OFFS = _offsets()


def _body(ti_hbm, tjs_hbm, h_hbm, emb_hbm, out_hbm,
          gtT_ref, hf_ref, ebuf_ref, obuf_ref, ti_ref, tjs_ref, h_ref,
          esems, osems, ssems):
    def ecopy(idx):
        off, sz = _CHUNK_OFFS[idx], _CHUNK_SIZES[idx]
        return pltpu.make_async_copy(
            emb_hbm.at[pl.ds(off, sz), :], ebuf_ref.at[pl.ds(off, sz), :],
            esems.at[idx])

    def ocopy(idx):
        off, sz = _CHUNK_OFFS[idx], _CHUNK_SIZES[idx]
        return pltpu.make_async_copy(
            obuf_ref.at[pl.ds(off, sz)], out_hbm.at[pl.ds(off, sz)],
            osems.at[idx])

    cp_ti = pltpu.make_async_copy(ti_hbm, ti_ref, ssems.at[0])
    cp_tjs = pltpu.make_async_copy(tjs_hbm, tjs_ref, ssems.at[1])
    cp_h = pltpu.make_async_copy(h_hbm, h_ref, ssems.at[2])
    cp_ti.start()
    cp_tjs.start()
    cp_h.start()
    for idx in range(len(_CHUNK_SIZES)):
        ecopy(idx).start()
    cp_ti.wait()
    cp_tjs.wait()
    gtT_ref[:] = jnp.exp(tjs_ref[0, :][:, None] - ti_ref[0, :][None, :])
    cp_h.wait()
    hf_ref[:] = h_ref[0].astype(jnp.float32)

    for idx in range(len(_CHUNK_SIZES)):
        off, sz = _CHUNK_OFFS[idx], _CHUNK_SIZES[idx]
        ecopy(idx).wait()
        P = jax.lax.dot_general(
            ebuf_ref[pl.ds(off, sz), :], hf_ref[:],
            (((1,), (1,)), ((), ())),
            preferred_element_type=jnp.float32)
        res = jnp.dot(P, gtT_ref[:], preferred_element_type=jnp.float32)
        obuf_ref[pl.ds(off, sz)] = res.reshape(sz, _SUB, _LANES)
        ocopy(idx).start()

    for idx in range(len(_CHUNK_SIZES)):
        ocopy(idx).wait()


def kernel(ti, tjs, ci, cjs, emb_weight):
    del ci
    ti_row = jnp.reshape(ti, (1, _BATCH))
    out = pl.pallas_call(
        _body,
        in_specs=[
            pl.BlockSpec(memory_space=pl.ANY),
            pl.BlockSpec(memory_space=pl.ANY),
            pl.BlockSpec(memory_space=pl.ANY),
            pl.BlockSpec(memory_space=pl.ANY),
        ],
        out_specs=pl.BlockSpec(memory_space=pl.ANY),
        out_shape=jax.ShapeDtypeStruct((_NUM_TYPE, _SUB, _LANES), jnp.float32),
        scratch_shapes=[
            pltpu.VMEM((_HIST, _BATCH), jnp.float32),
            pltpu.VMEM((_HIST, _NUM_TYPE), jnp.float32),
            pltpu.VMEM((_NUM_TYPE, _NUM_TYPE), jnp.float32),
            pltpu.VMEM((_NUM_TYPE, _SUB, _LANES), jnp.float32),
            pltpu.VMEM((1, _BATCH), jnp.float32),
            pltpu.VMEM((1, _HIST), jnp.float32),
            pltpu.VMEM((1, _HIST, _NUM_TYPE), jnp.int32),
            pltpu.SemaphoreType.DMA((len(_CHUNK_SIZES),)),
            pltpu.SemaphoreType.DMA((len(_CHUNK_SIZES),)),
            pltpu.SemaphoreType.DMA((3,)),
        ],
    )(ti_row, tjs, cjs, emb_weight)
    return jnp.reshape(out, (_NUM_TYPE, _BATCH, 1))


# speedup vs baseline: 1.4116x; 1.4116x over previous
import jax
import jax.numpy as jnp
from jax.experimental import pallas as pl
from jax.experimental.pallas import tpu as pltpu

_NUM_TYPE = 1000
_BATCH = 1024
_HIST = 200
_CHUNK_SIZES = (248, 248, 248, 256)
_LANES = 128
_SUB = _BATCH // _LANES


def _offsets():
    offs, o = [], 0
    for s in _CHUNK_SIZES:
        offs.append(o)
        o += s
    return tuple(offs)


_CHUNK_OFFS = _offsets()


def _body(ti_hbm, tjs_hbm, h_hbm, emb_hbm, out_hbm,
          gtT_ref, hf_ref, ebuf_ref, obuf_ref, ti_ref, tjs_ref, h_ref,
          esems, osems, ssems):
    def ecopy(idx):
        off, sz = _CHUNK_OFFS[idx], _CHUNK_SIZES[idx]
        return pltpu.make_async_copy(
            emb_hbm.at[pl.ds(off, sz), :], ebuf_ref.at[pl.ds(off, sz), :],
            esems.at[idx])

    def ocopy(idx):
        off, sz = _CHUNK_OFFS[idx], _CHUNK_SIZES[idx]
        return pltpu.make_async_copy(
            obuf_ref.at[pl.ds(off, sz)], out_hbm.at[pl.ds(off, sz)],
            osems.at[idx])

    cp_ti = pltpu.make_async_copy(ti_hbm, ti_ref, ssems.at[0])
    cp_tjs = pltpu.make_async_copy(tjs_hbm, tjs_ref, ssems.at[1])
    cp_h = pltpu.make_async_copy(h_hbm, h_ref, ssems.at[2])
    cp_ti.start()
    cp_tjs.start()
    cp_h.start()
    for idx in range(len(_CHUNK_SIZES)):
        ecopy(idx).start()
    cp_ti.wait()
    cp_tjs.wait()
    gtT_ref[:] = jnp.exp(tjs_ref[0, :][:, None] - ti_ref[0, :][None, :])
    cp_h.wait()
    hf_ref[:] = h_ref[0].astype(jnp.float32)

    for idx in range(len(_CHUNK_SIZES)):
        off, sz = _CHUNK_OFFS[idx], _CHUNK_SIZES[idx]
        ecopy(idx).wait()
        P = jax.lax.dot_general(
            ebuf_ref[pl.ds(off, sz), :], hf_ref[:],
            (((1,), (1,)), ((), ())),
            preferred_element_type=jnp.float32)
        res = jnp.dot(P, gtT_ref[:], preferred_element_type=jnp.float32)
        obuf_ref[pl.ds(off, sz)] = res.reshape(sz, _SUB, _LANES)
        ocopy(idx).start()

    for idx in range(len(_CHUNK_SIZES)):
        ocopy(idx).wait()


def kernel(ti, tjs, ci, cjs, emb_weight):
    del ci
    ti_row = jnp.reshape(ti, (1, _BATCH))
    out = pl.pallas_call(
        _body,
        in_specs=[
            pl.BlockSpec(memory_space=pl.ANY),
            pl.BlockSpec(memory_space=pl.ANY),
            pl.BlockSpec(memory_space=pl.ANY),
            pl.BlockSpec(memory_space=pl.ANY),
        ],
        out_specs=pl.BlockSpec(memory_space=pl.ANY),
        out_shape=jax.ShapeDtypeStruct((_NUM_TYPE, _SUB, _LANES), jnp.float32),
        scratch_shapes=[
            pltpu.VMEM((_HIST, _BATCH), jnp.float32),
            pltpu.VMEM((_HIST, _NUM_TYPE), jnp.float32),
            pltpu.VMEM((_NUM_TYPE, _NUM_TYPE), jnp.float32),
            pltpu.VMEM((_NUM_TYPE, _SUB, _LANES), jnp.float32),
            pltpu.VMEM((1, _BATCH), jnp.float32),
            pltpu.VMEM((1, _HIST), jnp.float32),
            pltpu.VMEM((1, _HIST, _NUM_TYPE), jnp.int32),
            pltpu.SemaphoreType.DMA((len(_CHUNK_SIZES),)),
            pltpu.SemaphoreType.DMA((len(_CHUNK_SIZES),)),
            pltpu.SemaphoreType.DMA((3,)),
        ],
    )(ti_row, tjs, cjs, emb_weight)
    return jnp.reshape(out, (_NUM_TYPE, _BATCH, 1))
